# Initial kernel scaffold; baseline (speedup 1.0000x reference)
import jax
import jax.numpy as jnp
from jax.experimental import pallas as pl


def kernel(xi, sparse, W_rk, b_rk, W_wk, b_wk, W_wv, b_wv, W_wg, b_wg):
    raise NotImplementedError("write your pallas kernel here")


# speedup vs baseline: 2.0587x; 2.0587x over previous
import jax
import jax.numpy as jnp
from jax.experimental import pallas as pl

_B, _M, _W, _R, _K = 32, 16384, 64, 8, 8


def _knn_body(rk_ref, sp_ref, m2_ref, rv_ref, d_ref):
    s = sp_ref[0]
    rk = rk_ref[0]
    m2 = m2_ref[0]
    k2 = jnp.sum(rk * rk, axis=1, keepdims=True)
    cross = jax.lax.dot_general(
        rk.astype(jnp.bfloat16), s.astype(jnp.bfloat16), (((1,), (1,)), ((), ())),
        preferred_element_type=jnp.float32)
    d2 = (k2 + m2) - 2.0 * cross
    iota = jax.lax.broadcasted_iota(jnp.int32, (_R, _M), 1)
    d = d2
    rows, vals = [], []
    for _ in range(_K):
        mv = jnp.min(d, axis=1, keepdims=True)
        cand = jnp.where(d == mv, iota, _M)
        mi = jnp.min(cand, axis=1, keepdims=True)
        sel = cand == mi
        row_k = jax.lax.dot_general(
            sel.astype(jnp.float32), s, (((1,), (0,)), ((), ())),
            precision=jax.lax.Precision.HIGHEST,
            preferred_element_type=jnp.float32)
        rows.append(row_k)
        vals.append(mv)
        d = jnp.where(sel, jnp.float32(jnp.inf), d)
    rv_ref[...] = jnp.stack(rows, axis=1).reshape(_R, 1, _K, _W)
    d_ref[...] = jnp.concatenate(vals, axis=1).reshape(1, _R, _K)


def _norm_body(d_ref, w_ref):
    d = d_ref[...]
    mk = jnp.max(d, axis=2, keepdims=True)
    mb = jnp.max(mk, axis=0, keepdims=True)
    w_ref[...] = d / mb


def kernel(xi, sparse, W_rk, b_rk, W_wk, b_wk, W_wv, b_wv, W_wg, b_wg):
    rk = (jax.lax.dot_general(
        xi.astype(jnp.bfloat16), W_rk.astype(jnp.bfloat16), (((1,), (0,)), ((), ())),
        preferred_element_type=jnp.float32) + b_rk).reshape(_B, _R, _W)
    m2 = jnp.sum(sparse ** 2, axis=-1).reshape(_B, 1, _M)

    rv, dists = pl.pallas_call(
        _knn_body,
        grid=(_B,),
        in_specs=[
            pl.BlockSpec((1, _R, _W), lambda b: (b, 0, 0)),
            pl.BlockSpec((1, _M, _W), lambda b: (b, 0, 0)),
            pl.BlockSpec((1, 1, _M), lambda b: (b, 0, 0)),
        ],
        out_specs=[
            pl.BlockSpec((_R, 1, _K, _W), lambda b: (0, b, 0, 0)),
            pl.BlockSpec((1, _R, _K), lambda b: (b, 0, 0)),
        ],
        out_shape=[
            jax.ShapeDtypeStruct((_R, _B, _K, _W), jnp.float32),
            jax.ShapeDtypeStruct((_B, _R, _K), jnp.float32),
        ],
    )(rk, sparse, m2)

    wts = pl.pallas_call(
        _norm_body,
        out_shape=jax.ShapeDtypeStruct((_B, _R, _K), jnp.float32),
    )(dists)
    return rv, jnp.transpose(wts, (1, 0, 2))


# speedup vs baseline: 3.9292x; 1.9086x over previous
import jax
import jax.numpy as jnp
from jax import lax
from jax.experimental import pallas as pl
from jax.experimental.pallas import tpu as pltpu
from jax.experimental.pallas import tpu_sc as plsc

_B, _M, _W, _R, _K = 32, 16384, 64, 8, 8

_INFO = plsc.get_sparse_core_info()
_NC, _NS = _INFO.num_cores, _INFO.num_subcores
_NWK = _NC * _NS
_NROW = _R * _B * _K
_JPW = _NROW // _NWK


def _knn_body(rk_ref, sp_ref, m2_ref, d_ref, i_ref):
    s = sp_ref[0]
    rk = rk_ref[0]
    m2 = m2_ref[0]
    b = pl.program_id(0)
    k2 = jnp.sum(rk * rk, axis=1, keepdims=True)
    cross = jax.lax.dot_general(
        rk.astype(jnp.bfloat16), s.astype(jnp.bfloat16), (((1,), (1,)), ((), ())),
        preferred_element_type=jnp.float32)
    d2 = (k2 + m2) - 2.0 * cross
    iota = jax.lax.broadcasted_iota(jnp.int32, (_R, _M), 1)
    d = d2
    vals, idxs = [], []
    for _ in range(_K):
        mv = jnp.min(d, axis=1, keepdims=True)
        cand = jnp.where(d == mv, iota, _M)
        mi = jnp.min(cand, axis=1, keepdims=True)
        vals.append(mv)
        idxs.append(mi)
        d = jnp.where(cand == mi, jnp.float32(jnp.inf), d)
    d_ref[...] = jnp.concatenate(vals, axis=1).reshape(1, _R, _K)
    i_ref[...] = (jnp.concatenate(idxs, axis=1) + b * _M).reshape(1, _R, _K)


def _norm_body(d_ref, w_ref):
    d = d_ref[...]
    mk = jnp.max(d, axis=2, keepdims=True)
    mb = jnp.max(mk, axis=0, keepdims=True)
    w_ref[...] = d / mb


def _gather_body(fidx_hbm, sp_hbm, out_hbm, fid_v, rows_v, sem):
    wid = lax.axis_index("s") * _NC + lax.axis_index("c")
    pltpu.sync_copy(fidx_hbm.at[pl.ds(wid * _JPW, _JPW)], fid_v)
    pltpu.async_copy(sp_hbm.at[fid_v], rows_v, sem).wait()
    for r in range(_R):
        pltpu.sync_copy(rows_v.at[pl.ds(r * _K, _K)],
                        out_hbm.at[pl.ds(r * (_B * _K) + wid * _K, _K)])


def _half_body(x_ref, p_ref, o_ref):
    x = x_ref[...]
    p = p_ref[...]
    o_ref[...] = jnp.where(p != 0, x[:, _W:], x[:, :_W])


def kernel(xi, sparse, W_rk, b_rk, W_wk, b_wk, W_wv, b_wv, W_wg, b_wg):
    rk = (jax.lax.dot_general(
        xi.astype(jnp.bfloat16), W_rk.astype(jnp.bfloat16), (((1,), (0,)), ((), ())),
        preferred_element_type=jnp.float32) + b_rk).reshape(_B, _R, _W)
    m2 = jnp.sum(sparse ** 2, axis=-1).reshape(_B, 1, _M)

    dists, fidx = pl.pallas_call(
        _knn_body,
        grid=(_B,),
        in_specs=[
            pl.BlockSpec((1, _R, _W), lambda b: (b, 0, 0)),
            pl.BlockSpec((1, _M, _W), lambda b: (b, 0, 0)),
            pl.BlockSpec((1, 1, _M), lambda b: (b, 0, 0)),
        ],
        out_specs=[
            pl.BlockSpec((1, _R, _K), lambda b: (b, 0, 0)),
            pl.BlockSpec((1, _R, _K), lambda b: (b, 0, 0)),
        ],
        out_shape=[
            jax.ShapeDtypeStruct((_B, _R, _K), jnp.float32),
            jax.ShapeDtypeStruct((_B, _R, _K), jnp.int32),
        ],
    )(rk, sparse, m2)

    gcall = pl.kernel(
        _gather_body,
        mesh=plsc.VectorSubcoreMesh(core_axis_name="c", subcore_axis_name="s"),
        out_type=jax.ShapeDtypeStruct((_NROW, 2 * _W), jnp.float32),
        scratch_types=[
            pltpu.VMEM((_JPW,), jnp.int32),
            pltpu.VMEM((_JPW, 2 * _W), jnp.float32),
            pltpu.SemaphoreType.DMA,
        ],
    )
    pairs = gcall((fidx >> 1).reshape(_NROW), sparse.reshape(_B * _M // 2, 2 * _W))
    parity = jnp.transpose((fidx & 1).reshape(_B, _R, _K), (1, 0, 2)).reshape(_NROW, 1)
    rv = pl.pallas_call(
        _half_body,
        out_shape=jax.ShapeDtypeStruct((_NROW, _W), jnp.float32),
    )(pairs, parity).reshape(_R, _B, _K, _W)

    wts = pl.pallas_call(
        _norm_body,
        out_shape=jax.ShapeDtypeStruct((_B, _R, _K), jnp.float32),
    )(dists)
    return rv, jnp.transpose(wts, (1, 0, 2))
